# baseline (speedup 1.0000x reference)
import functools

import jax
import jax.numpy as jnp
from jax import lax
from jax.experimental import pallas as pl
from jax.experimental.pallas import tpu as pltpu
from jax.experimental.pallas import tpu_sc as plsc

QR = 1000
BATCH = 16384
HIST = 20
DIM = 64
HDIM = DIM // 2
RSTRIDE = 33
LANES = 16

NUM_CORES = 2
NUM_SUBCORES = 16
NUM_WORKERS = NUM_CORES * NUM_SUBCORES
NUM_GROUPS = NUM_WORKERS // 2
BAGS_PER_GROUP = BATCH // NUM_GROUPS
IDS_PER_GROUP = BAGS_PER_GROUP * HIST
NB = 64
NCHUNK = BAGS_PER_GROUP // NB
NPAIR = NCHUNK // 2
NBLK = NB // LANES
CGRP = 8
NCG = HDIM // CGRP


@functools.partial(
    pl.kernel,
    mesh=plsc.VectorSubcoreMesh(core_axis_name="c", subcore_axis_name="s"),
    out_type=jax.ShapeDtypeStruct((2 * BATCH * HDIM,), jnp.float32),
    compiler_params=pltpu.CompilerParams(needs_layout_passes=False),
    scratch_types=[
        pltpu.VMEM((QR * RSTRIDE,), jnp.float32),
        pltpu.VMEM((QR * RSTRIDE,), jnp.float32),
        pltpu.VMEM((IDS_PER_GROUP,), jnp.int32),
        pltpu.VMEM((HIST * LANES,), jnp.int32),
        pltpu.VMEM((HIST * LANES,), jnp.int32),
        pltpu.VMEM((NB * HDIM,), jnp.float32),
        pltpu.VMEM((NB * HDIM,), jnp.float32),
        pltpu.SemaphoreType.DMA,
        pltpu.SemaphoreType.DMA,
        pltpu.SemaphoreType.DMA,
    ],
)
def _qr_bag(idx_hbm, wq_hbm, wr_hbm, out_hbm, wq_v, wr_v, idx_v,
            offq_v, offr_v, out0, out1, sem_o0, sem_o1, sem_t):
    wid = lax.axis_index("s") * NUM_CORES + lax.axis_index("c")
    half = wid % 2
    group = wid // 2
    base = group * BAGS_PER_GROUP

    def out_slice(chunk):
        return out_hbm.at[
            pl.ds(half * (BATCH * HDIM) + (base + chunk * NB) * HDIM, NB * HDIM)
        ]

    lane_iota = lax.iota(jnp.int32, LANES)
    scatter_base = lane_iota * HDIM

    def compute_chunk(chunk, out_v):
        def block_body(b, carry):
            bag0 = chunk * NB + b * LANES
            for h in range(HIST):
                ids = idx_v[pl.ds(h * BAGS_PER_GROUP + bag0, LANES)]
                q = (ids.astype(jnp.float32) * jnp.float32(0.001)).astype(
                    jnp.int32)
                r = ids - q * QR
                q = jnp.where(r < 0, q - 1, q)
                r = jnp.where(r < 0, r + QR, r)
                q = jnp.where(r >= QR, q + 1, q)
                r = jnp.where(r >= QR, r - QR, r)
                offq_v[pl.ds(h * LANES, LANES)] = q * RSTRIDE
                offr_v[pl.ds(h * LANES, LANES)] = r * RSTRIDE

            def cg_body(cg, carry2):
                col0 = cg * CGRP
                accq = [jnp.zeros((LANES,), jnp.float32) for _ in range(CGRP)]
                accr = [jnp.zeros((LANES,), jnp.float32) for _ in range(CGRP)]
                for h in range(HIST):
                    qo = offq_v[pl.ds(h * LANES, LANES)]
                    ro = offr_v[pl.ds(h * LANES, LANES)]
                    for j in range(CGRP):
                        accq[j] = accq[j] + plsc.load_gather(
                            wq_v, [qo + (col0 + j)])
                        accr[j] = accr[j] + plsc.load_gather(
                            wr_v, [ro + (col0 + j)])
                dst = scatter_base + (b * LANES * HDIM + col0)
                for j in range(CGRP):
                    plsc.store_scatter(out_v, [dst + j], accq[j] * accr[j])
                return carry2

            lax.fori_loop(0, NCG, cg_body, 0)
            return carry

        lax.fori_loop(0, NBLK, block_body, 0)

    pltpu.async_copy(idx_hbm.at[pl.ds(group * IDS_PER_GROUP, IDS_PER_GROUP)],
                     idx_v, sem_t)
    pltpu.async_copy(wq_hbm.at[pl.ds(half * (QR * RSTRIDE), QR * RSTRIDE)],
                     wq_v, sem_t)
    pltpu.async_copy(wr_hbm.at[pl.ds(half * (QR * RSTRIDE), QR * RSTRIDE)],
                     wr_v, sem_t)
    pltpu.make_async_copy(idx_hbm.at[pl.ds(0, IDS_PER_GROUP)], idx_v, sem_t).wait()
    pltpu.make_async_copy(wq_hbm.at[pl.ds(0, QR * RSTRIDE)], wq_v, sem_t).wait()
    pltpu.make_async_copy(wr_hbm.at[pl.ds(0, QR * RSTRIDE)], wr_v, sem_t).wait()

    def pair_of_chunks(j, carry):
        ca = 2 * j
        cb = 2 * j + 1

        @pl.when(j > 0)
        def _():
            pltpu.make_async_copy(out0, out_slice(ca), sem_o0).wait()

        compute_chunk(ca, out0)
        pltpu.async_copy(out0, out_slice(ca), sem_o0)

        @pl.when(j > 0)
        def _():
            pltpu.make_async_copy(out1, out_slice(cb), sem_o1).wait()

        compute_chunk(cb, out1)
        pltpu.async_copy(out1, out_slice(cb), sem_o1)
        return carry

    lax.fori_loop(0, NPAIR, pair_of_chunks, 0)

    pltpu.make_async_copy(out0, out_slice(NCHUNK - 2), sem_o0).wait()
    pltpu.make_async_copy(out1, out_slice(NCHUNK - 1), sem_o1).wait()


def kernel(input_, quotient_embed_weight, remainder_embed_weight):
    def halves(w):
        w3 = w.reshape(QR, 2, HDIM).transpose(1, 0, 2)
        w3 = jnp.pad(w3, ((0, 0), (0, 0), (0, RSTRIDE - HDIM)))
        return w3.reshape(-1)

    ids_t = input_.reshape(NUM_GROUPS, BAGS_PER_GROUP, HIST)
    ids_t = ids_t.transpose(0, 2, 1).reshape(-1)

    out = _qr_bag(
        ids_t,
        halves(quotient_embed_weight),
        halves(remainder_embed_weight),
    )
    return out.reshape(2, BATCH, HDIM).transpose(1, 0, 2).reshape(BATCH, DIM)


# speedup vs baseline: 1.7978x; 1.7978x over previous
import functools

import jax
import jax.numpy as jnp
from jax import lax
from jax.experimental import pallas as pl
from jax.experimental.pallas import tpu as pltpu
from jax.experimental.pallas import tpu_sc as plsc

QR = 1000
BATCH = 16384
HIST = 20
DIM = 64
HDIM = DIM // 2
LANES = 16

NUM_CORES = 2
NUM_SUBCORES = 16
NUM_WORKERS = NUM_CORES * NUM_SUBCORES
NUM_GROUPS = NUM_WORKERS // 2
BAGS_PER_GROUP = BATCH // NUM_GROUPS
IDS_PER_GROUP = BAGS_PER_GROUP * HIST
NB = 64
NCHUNK = BAGS_PER_GROUP // NB
NPAIR = NCHUNK // 2
QUAD = 4
IDS_PER_QUAD = QUAD * HIST


@functools.partial(
    pl.kernel,
    mesh=plsc.VectorSubcoreMesh(core_axis_name="c", subcore_axis_name="s"),
    out_type=jax.ShapeDtypeStruct((BATCH, DIM), jnp.float32),
    compiler_params=pltpu.CompilerParams(use_tc_tiling_on_sc=False),
    scratch_types=[
        pltpu.VMEM((QR * HDIM,), jnp.float32),
        pltpu.VMEM((QR * HDIM,), jnp.float32),
        pltpu.VMEM((IDS_PER_GROUP,), jnp.int32),
        pltpu.VMEM((NB, HDIM), jnp.float32),
        pltpu.VMEM((NB, HDIM), jnp.float32),
        pltpu.SemaphoreType.DMA,
        pltpu.SemaphoreType.DMA,
        pltpu.SemaphoreType.DMA,
    ],
)
def _qr_bag(idx_hbm, wq_hbm, wr_hbm, out_hbm, wq_v, wr_v, idx_v,
            out0, out1, sem_o0, sem_o1, sem_t):
    wid = lax.axis_index("s") * NUM_CORES + lax.axis_index("c")
    half = wid % 2
    group = wid // 2
    base = group * BAGS_PER_GROUP

    def out_slice(chunk):
        return out_hbm.at[
            pl.ds(base + chunk * NB, NB), pl.ds(half * HDIM, HDIM)
        ]

    def compute_chunk(chunk, out_v):
        def quad_body(p, carry2):
            ib = chunk * (NB * HIST) + p * IDS_PER_QUAD
            packv = []
            for t in range(IDS_PER_QUAD // LANES):
                ids = idx_v[pl.ds(ib + t * LANES, LANES)]
                q = (ids.astype(jnp.float32) * jnp.float32(0.001)).astype(
                    jnp.int32)
                r = ids - q * QR
                q = jnp.where(r < 0, q - 1, q)
                r = jnp.where(r < 0, r + QR, r)
                q = jnp.where(r >= QR, q + 1, q)
                r = jnp.where(r >= QR, r - QR, r)
                packv.append((q * (HDIM << 16)) + r * HDIM)
            for s in range(QUAD):
                acc = [jnp.zeros((LANES,), jnp.float32) for _ in range(4)]
                for h in range(HIST):
                    g = s * HIST + h
                    pk = packv[g // LANES][g % LANES]
                    qoff = lax.shift_right_logical(pk, 16)
                    roff = jnp.bitwise_and(pk, 0xFFFF)
                    acc[0] = acc[0] + wq_v[pl.ds(qoff, LANES)]
                    acc[1] = acc[1] + wq_v[pl.ds(qoff + LANES, LANES)]
                    acc[2] = acc[2] + wr_v[pl.ds(roff, LANES)]
                    acc[3] = acc[3] + wr_v[pl.ds(roff + LANES, LANES)]
                b = p * QUAD + s
                out_v[b, pl.ds(0, LANES)] = acc[0] * acc[2]
                out_v[b, pl.ds(LANES, LANES)] = acc[1] * acc[3]
            return carry2

        lax.fori_loop(0, NB // QUAD, quad_body, 0)

    pltpu.async_copy(idx_hbm.at[pl.ds(base * HIST, IDS_PER_GROUP)], idx_v, sem_t)
    pltpu.async_copy(wq_hbm.at[pl.ds(half * (QR * HDIM), QR * HDIM)], wq_v, sem_t)
    pltpu.async_copy(wr_hbm.at[pl.ds(half * (QR * HDIM), QR * HDIM)], wr_v, sem_t)
    pltpu.make_async_copy(idx_hbm.at[pl.ds(0, IDS_PER_GROUP)], idx_v, sem_t).wait()
    pltpu.make_async_copy(wq_hbm.at[pl.ds(0, QR * HDIM)], wq_v, sem_t).wait()
    pltpu.make_async_copy(wr_hbm.at[pl.ds(0, QR * HDIM)], wr_v, sem_t).wait()

    def pair_of_chunks(j, carry):
        ca = 2 * j
        cb = 2 * j + 1

        @pl.when(j > 0)
        def _():
            pltpu.make_async_copy(out0, out_slice(ca), sem_o0).wait()

        compute_chunk(ca, out0)
        pltpu.async_copy(out0, out_slice(ca), sem_o0)

        @pl.when(j > 0)
        def _():
            pltpu.make_async_copy(out1, out_slice(cb), sem_o1).wait()

        compute_chunk(cb, out1)
        pltpu.async_copy(out1, out_slice(cb), sem_o1)
        return carry

    lax.fori_loop(0, NPAIR, pair_of_chunks, 0)

    pltpu.make_async_copy(out0, out_slice(NCHUNK - 2), sem_o0).wait()
    pltpu.make_async_copy(out1, out_slice(NCHUNK - 1), sem_o1).wait()


def kernel(input_, quotient_embed_weight, remainder_embed_weight):
    def halves(w):
        return w.reshape(QR, 2, HDIM).transpose(1, 0, 2).reshape(-1)

    return _qr_bag(
        input_.reshape(-1),
        halves(quotient_embed_weight),
        halves(remainder_embed_weight),
    )


# speedup vs baseline: 1.8000x; 1.0012x over previous
import functools

import jax
import jax.numpy as jnp
from jax import lax
from jax.experimental import pallas as pl
from jax.experimental.pallas import tpu as pltpu
from jax.experimental.pallas import tpu_sc as plsc

QR = 1000
BATCH = 16384
HIST = 20
DIM = 64
HDIM = DIM // 2
LANES = 16

NUM_CORES = 2
NUM_SUBCORES = 16
NUM_WORKERS = NUM_CORES * NUM_SUBCORES
NUM_GROUPS = NUM_WORKERS // 2
BAGS_PER_GROUP = BATCH // NUM_GROUPS
IDS_PER_GROUP = BAGS_PER_GROUP * HIST
NB = 128
NCHUNK = BAGS_PER_GROUP // NB
NPAIR = NCHUNK // 2
QUAD = 4
IDS_PER_QUAD = QUAD * HIST


@functools.partial(
    pl.kernel,
    mesh=plsc.VectorSubcoreMesh(core_axis_name="c", subcore_axis_name="s"),
    out_type=jax.ShapeDtypeStruct((BATCH, DIM), jnp.float32),
    compiler_params=pltpu.CompilerParams(use_tc_tiling_on_sc=False),
    scratch_types=[
        pltpu.VMEM((QR * HDIM,), jnp.float32),
        pltpu.VMEM((QR * HDIM,), jnp.float32),
        pltpu.VMEM((IDS_PER_GROUP,), jnp.int32),
        pltpu.VMEM((NB, HDIM), jnp.float32),
        pltpu.VMEM((NB, HDIM), jnp.float32),
        pltpu.SemaphoreType.DMA,
        pltpu.SemaphoreType.DMA,
        pltpu.SemaphoreType.DMA,
    ],
)
def _qr_bag(idx_hbm, wq_hbm, wr_hbm, out_hbm, wq_v, wr_v, idx_v,
            out0, out1, sem_o0, sem_o1, sem_t):
    wid = lax.axis_index("s") * NUM_CORES + lax.axis_index("c")
    half = wid % 2
    group = wid // 2
    base = group * BAGS_PER_GROUP

    def out_slice(chunk):
        return out_hbm.at[
            pl.ds(base + chunk * NB, NB), pl.ds(half * HDIM, HDIM)
        ]

    def compute_chunk(chunk, out_v):
        def quad_body(p, carry2):
            ib = chunk * (NB * HIST) + p * IDS_PER_QUAD
            packv = []
            for t in range(IDS_PER_QUAD // LANES):
                ids = idx_v[pl.ds(ib + t * LANES, LANES)]
                q = (ids.astype(jnp.float32) * jnp.float32(0.001)).astype(
                    jnp.int32)
                r = ids - q * QR
                q = jnp.where(r < 0, q - 1, q)
                r = jnp.where(r < 0, r + QR, r)
                q = jnp.where(r >= QR, q + 1, q)
                r = jnp.where(r >= QR, r - QR, r)
                packv.append((q * (HDIM << 16)) + r * HDIM)
            for s in range(QUAD):
                acc = [jnp.zeros((LANES,), jnp.float32) for _ in range(4)]
                for h in range(HIST):
                    g = s * HIST + h
                    pk = packv[g // LANES][g % LANES]
                    qoff = lax.shift_right_logical(pk, 16)
                    roff = jnp.bitwise_and(pk, 0xFFFF)
                    acc[0] = acc[0] + wq_v[pl.ds(qoff, LANES)]
                    acc[1] = acc[1] + wq_v[pl.ds(qoff + LANES, LANES)]
                    acc[2] = acc[2] + wr_v[pl.ds(roff, LANES)]
                    acc[3] = acc[3] + wr_v[pl.ds(roff + LANES, LANES)]
                b = p * QUAD + s
                out_v[b, pl.ds(0, LANES)] = acc[0] * acc[2]
                out_v[b, pl.ds(LANES, LANES)] = acc[1] * acc[3]
            return carry2

        lax.fori_loop(0, NB // QUAD, quad_body, 0)

    pltpu.async_copy(idx_hbm.at[pl.ds(base * HIST, IDS_PER_GROUP)], idx_v, sem_t)
    pltpu.async_copy(wq_hbm.at[pl.ds(half * (QR * HDIM), QR * HDIM)], wq_v, sem_t)
    pltpu.async_copy(wr_hbm.at[pl.ds(half * (QR * HDIM), QR * HDIM)], wr_v, sem_t)
    pltpu.make_async_copy(idx_hbm.at[pl.ds(0, IDS_PER_GROUP)], idx_v, sem_t).wait()
    pltpu.make_async_copy(wq_hbm.at[pl.ds(0, QR * HDIM)], wq_v, sem_t).wait()
    pltpu.make_async_copy(wr_hbm.at[pl.ds(0, QR * HDIM)], wr_v, sem_t).wait()

    def pair_of_chunks(j, carry):
        ca = 2 * j
        cb = 2 * j + 1

        @pl.when(j > 0)
        def _():
            pltpu.make_async_copy(out0, out_slice(ca), sem_o0).wait()

        compute_chunk(ca, out0)
        pltpu.async_copy(out0, out_slice(ca), sem_o0)

        @pl.when(j > 0)
        def _():
            pltpu.make_async_copy(out1, out_slice(cb), sem_o1).wait()

        compute_chunk(cb, out1)
        pltpu.async_copy(out1, out_slice(cb), sem_o1)
        return carry

    lax.fori_loop(0, NPAIR, pair_of_chunks, 0)

    pltpu.make_async_copy(out0, out_slice(NCHUNK - 2), sem_o0).wait()
    pltpu.make_async_copy(out1, out_slice(NCHUNK - 1), sem_o1).wait()


def kernel(input_, quotient_embed_weight, remainder_embed_weight):
    def halves(w):
        return w.reshape(QR, 2, HDIM).transpose(1, 0, 2).reshape(-1)

    return _qr_bag(
        input_.reshape(-1),
        halves(quotient_embed_weight),
        halves(remainder_embed_weight),
    )
